# baseline (speedup 1.0000x reference)
import functools

import jax
import jax.numpy as jnp
from jax import lax
from jax.experimental import pallas as pl
from jax.experimental.pallas import tpu as pltpu
from jax.experimental.pallas import tpu_sc as plsc

_B = 16384
_K = 128
_H = 256
_NC = 2
_NS = 16
_NW = _NC * _NS
_BPW = _B // _NW
_CHUNK = 128
_NCHUNK = _BPW // _CHUNK


def _sc_body(uid_hbm, iid_hbm, w_hbm, u_hbm, w1_hbm, w2_hbm, b1_hbm, b2_hbm,
             out_hbm,
             uid_v, iid_v, wbuf0, ubuf0, wbuf1, ubuf1, wbuf2, ubuf2,
             obuf, vbuf, w1buf, w2buf, b1buf, b2buf, pvbuf, gathbuf, shared,
             sem_w0, sem_u0, sem_w1, sem_u1, sem_w2, sem_u2, sem_p):
    cid = lax.axis_index("c")
    sid = lax.axis_index("s")
    wid = sid * _NC + cid
    base = wid * _BPW
    c_uid = pltpu.async_copy(uid_hbm.at[pl.ds(base, _BPW)], uid_v, sem_w2)
    c_iid = pltpu.async_copy(iid_hbm.at[pl.ds(base, _BPW)], iid_v, sem_u2)
    c_w1 = pltpu.async_copy(w1_hbm.at[pl.ds(16 * sid, 16)], w1buf, sem_p)
    c_w2 = pltpu.async_copy(w2_hbm, w2buf, sem_p)
    c_b1 = pltpu.async_copy(b1_hbm, b1buf, sem_p)
    c_b2 = pltpu.async_copy(b2_hbm, b2buf.at[pl.ds(0, 1)], sem_p)
    c_uid.wait()
    c_iid.wait()

    lanes = lax.iota(jnp.int32, 16)
    perms = [lanes ^ d for d in (1, 2, 4, 8)]
    masks = [(lanes & d) == 0 for d in (1, 2, 4, 8)]
    dnums = lax.GatherDimensionNumbers(
        offset_dims=(), collapsed_slice_dims=(0,), start_index_map=(0,))

    def shuf(x, idx):
        return lax.gather(x, idx[:, None], dnums, (1,),
                          mode=lax.GatherScatterMode.PROMISE_IN_BOUNDS)

    bufs = [(wbuf0, ubuf0, sem_w0, sem_u0), (wbuf1, ubuf1, sem_w1, sem_u1),
            (wbuf2, ubuf2, sem_w2, sem_u2)]

    def fire(g):
        wb, ub, sw, su = bufs[g % 3]
        cw = pltpu.async_copy(
            w_hbm.at[uid_v.at[pl.ds(g * _CHUNK, _CHUNK)]], wb, sw)
        cu = pltpu.async_copy(
            u_hbm.at[iid_v.at[pl.ds(g * _CHUNK, _CHUNK)]], ub, su)
        return cw, cu

    pending = [fire(0), fire(1), fire(2)]

    c_w1.wait()
    c_w2.wait()
    c_b1.wait()
    c_b2.wait()
    w2loc = w2buf[pl.ds(16 * sid, 16)]
    for jb in range(16):
        acc = w1buf[0, pl.ds(16 * jb, 16)] * w2loc[0]
        for hh in range(1, 16):
            acc = acc + w1buf[hh, pl.ds(16 * jb, 16)] * w2loc[hh]
        pvbuf[pl.ds(16 * jb, 16)] = acc
    pltpu.sync_copy(pvbuf, shared.at[sid])
    cacc = w2buf[pl.ds(0, 16)] * b1buf[pl.ds(0, 16)]
    for jb in range(1, 16):
        cacc = cacc + w2buf[pl.ds(16 * jb, 16)] * b1buf[pl.ds(16 * jb, 16)]
    for p in perms:
        cacc = cacc + shuf(cacc, p)
    cval = cacc[0] + b2buf[pl.ds(0, 16)][0]
    plsc.subcore_barrier()
    pltpu.sync_copy(shared, gathbuf)
    for jb in range(16):
        vsum = gathbuf[0, pl.ds(16 * jb, 16)]
        for s in range(1, 16):
            vsum = vsum + gathbuf[s, pl.ds(16 * jb, 16)]
        vbuf[pl.ds(16 * jb, 16)] = vsum

    def compute(g):
        wb, ub, _, _ = bufs[g % 3]

        def group(t, inner):
            def jstep(j, accs):
                cw = vbuf[pl.ds(16 * j, 16)]
                cu = vbuf[pl.ds(128 + 16 * j, 16)]
                return tuple(
                    accs[i]
                    + wb[t * 16 + i, pl.ds(16 * j, 16)] * cw
                    + ub[t * 16 + i, pl.ds(16 * j, 16)] * cu
                    for i in range(16))

            zero = jnp.zeros((16,), jnp.float32)
            accs = lax.fori_loop(0, 8, jstep, (zero,) * 16)

            vals = list(accs)
            for p, m in zip(perms, masks):
                vals = [jnp.where(m, a + shuf(a, p), b + shuf(b, p))
                        for a, b in zip(vals[::2], vals[1::2])]
            outv = jnp.clip(vals[0] + cval, 0.5, 5.0)
            obuf[pl.ds(g * _CHUNK + t * 16, 16)] = outv
            return inner

        lax.fori_loop(0, _CHUNK // 16, group, 0)

    outs = []
    for g in range(_NCHUNK):
        cw, cu = pending[g]
        cw.wait()
        cu.wait()
        compute(g)
        if g + 3 < _NCHUNK:
            pending.append(fire(g + 3))
        outs.append(pltpu.async_copy(
            obuf.at[pl.ds(g * _CHUNK, _CHUNK)],
            out_hbm.at[pl.ds(base + g * _CHUNK, _CHUNK)], sem_p))
    for c in outs:
        c.wait()


def kernel(user_ids, item_ids, W, U, W1, b1, W2, b2):
    uid = user_ids.astype(jnp.int32)
    iid = item_ids.astype(jnp.int32)

    sc = functools.partial(
        pl.kernel,
        mesh=plsc.VectorSubcoreMesh(core_axis_name="c", subcore_axis_name="s"),
        out_type=jax.ShapeDtypeStruct((_B,), jnp.float32),
        scratch_types=[
            pltpu.VMEM((_BPW,), jnp.int32),
            pltpu.VMEM((_BPW,), jnp.int32),
            pltpu.VMEM((_CHUNK, _K), jnp.float32),
            pltpu.VMEM((_CHUNK, _K), jnp.float32),
            pltpu.VMEM((_CHUNK, _K), jnp.float32),
            pltpu.VMEM((_CHUNK, _K), jnp.float32),
            pltpu.VMEM((_CHUNK, _K), jnp.float32),
            pltpu.VMEM((_CHUNK, _K), jnp.float32),
            pltpu.VMEM((_BPW,), jnp.float32),
            pltpu.VMEM((_H,), jnp.float32),
            pltpu.VMEM((16, _H), jnp.float32),
            pltpu.VMEM((_H,), jnp.float32),
            pltpu.VMEM((_H,), jnp.float32),
            pltpu.VMEM((16,), jnp.float32),
            pltpu.VMEM((_H,), jnp.float32),
            pltpu.VMEM((16, _H), jnp.float32),
            pltpu.VMEM_SHARED((16, _H), jnp.float32),
            pltpu.SemaphoreType.DMA,
            pltpu.SemaphoreType.DMA,
            pltpu.SemaphoreType.DMA,
            pltpu.SemaphoreType.DMA,
            pltpu.SemaphoreType.DMA,
            pltpu.SemaphoreType.DMA,
            pltpu.SemaphoreType.DMA,
        ],
    )(_sc_body)
    return sc(uid, iid, W, U, W1, W2.reshape(_H), b1, b2)


# speedup vs baseline: 1.0631x; 1.0631x over previous
import functools

import jax
import jax.numpy as jnp
from jax import lax
from jax.experimental import pallas as pl
from jax.experimental.pallas import tpu as pltpu
from jax.experimental.pallas import tpu_sc as plsc

_B = 16384
_K = 128
_H = 256
_NC = 2
_NS = 16
_NW = _NC * _NS
_BPW = _B // _NW
_CHUNK = 128
_NCHUNK = _BPW // _CHUNK


def _vc_body(w1_ref, w2_ref, b1_ref, b2_ref, out_ref):
    v = jnp.dot(w2_ref[...], w1_ref[...], preferred_element_type=jnp.float32)
    c = jnp.sum(w2_ref[...] * b1_ref[...]) + b2_ref[0, 0]
    out_ref[pl.ds(0, 256)] = v[0]
    out_ref[pl.ds(256, 128)] = jnp.full((128,), c, jnp.float32)


def _sc_body(uid_hbm, iid_hbm, w_hbm, u_hbm, vc_hbm, out_hbm,
             uid_v, iid_v, wbuf0, ubuf0, wbuf1, ubuf1, wbuf2, ubuf2,
             obuf, vbuf,
             sem_w0, sem_u0, sem_w1, sem_u1, sem_w2, sem_u2, sem_p):
    wid = lax.axis_index("s") * _NC + lax.axis_index("c")
    base = wid * _BPW
    c_uid = pltpu.async_copy(uid_hbm.at[pl.ds(base, _BPW)], uid_v, sem_w2)
    c_iid = pltpu.async_copy(iid_hbm.at[pl.ds(base, _BPW)], iid_v, sem_u2)
    c_vc = pltpu.async_copy(vc_hbm, vbuf, sem_p)
    c_uid.wait()
    c_iid.wait()
    lanes = lax.iota(jnp.int32, 16)
    perms = [lanes ^ d for d in (1, 2, 4, 8)]
    masks = [(lanes & d) == 0 for d in (1, 2, 4, 8)]
    dnums = lax.GatherDimensionNumbers(
        offset_dims=(), collapsed_slice_dims=(0,), start_index_map=(0,))

    def shuf(x, idx):
        return lax.gather(x, idx[:, None], dnums, (1,),
                          mode=lax.GatherScatterMode.PROMISE_IN_BOUNDS)

    bufs = [(wbuf0, ubuf0, sem_w0, sem_u0), (wbuf1, ubuf1, sem_w1, sem_u1),
            (wbuf2, ubuf2, sem_w2, sem_u2)]

    chunks = [(0, 128), (128, 128), (256, 128), (384, 64), (448, 64)]

    def fire(k):
        off, size = chunks[k]
        wb, ub, sw, su = bufs[k % 3]
        cw = pltpu.async_copy(
            w_hbm.at[uid_v.at[pl.ds(off, size)]], wb.at[pl.ds(0, size)], sw)
        cu = pltpu.async_copy(
            u_hbm.at[iid_v.at[pl.ds(off, size)]], ub.at[pl.ds(0, size)], su)
        return cw, cu

    def compute(k):
        off, size = chunks[k]
        wb, ub, _, _ = bufs[k % 3]

        def group(t, inner):
            def jstep(j, accs):
                cw = vbuf[pl.ds(16 * j, 16)]
                cu = vbuf[pl.ds(128 + 16 * j, 16)]
                return tuple(
                    accs[i]
                    + wb[t * 16 + i, pl.ds(16 * j, 16)] * cw
                    + ub[t * 16 + i, pl.ds(16 * j, 16)] * cu
                    for i in range(16))

            zero = jnp.zeros((16,), jnp.float32)
            accs = lax.fori_loop(0, 8, jstep, (zero,) * 16)

            vals = list(accs)
            for p, m in zip(perms, masks):
                vals = [jnp.where(m, a + shuf(a, p), b + shuf(b, p))
                        for a, b in zip(vals[::2], vals[1::2])]
            outv = jnp.clip(vals[0] + cval, 0.5, 5.0)
            obuf[pl.ds(off + t * 16, 16)] = outv
            return inner

        lax.fori_loop(0, size // 16, group, 0)

    pending = [fire(0), fire(1), fire(2)]
    c_vc.wait()
    cval = vbuf[pl.ds(256, 16)][0]
    outs = []
    for k in range(len(chunks)):
        cw, cu = pending[k]
        cw.wait()
        cu.wait()
        compute(k)
        if k + 3 < len(chunks):
            pending.append(fire(k + 3))
        off, size = chunks[k]
        outs.append(pltpu.async_copy(
            obuf.at[pl.ds(off, size)],
            out_hbm.at[pl.ds(base + off, size)], sem_p))
    for c in outs:
        c.wait()


def kernel(user_ids, item_ids, W, U, W1, b1, W2, b2):
    uid = user_ids.astype(jnp.int32)
    iid = item_ids.astype(jnp.int32)

    vc_flat = pl.pallas_call(
        _vc_body,
        out_shape=jax.ShapeDtypeStruct((384,), jnp.float32),
    )(W1, W2, b1.reshape(1, _H), b2.reshape(1, 1))

    sc = functools.partial(
        pl.kernel,
        mesh=plsc.VectorSubcoreMesh(core_axis_name="c", subcore_axis_name="s"),
        out_type=jax.ShapeDtypeStruct((_B,), jnp.float32),
        scratch_types=[
            pltpu.VMEM((_BPW,), jnp.int32),
            pltpu.VMEM((_BPW,), jnp.int32),
            pltpu.VMEM((_CHUNK, _K), jnp.float32),
            pltpu.VMEM((_CHUNK, _K), jnp.float32),
            pltpu.VMEM((_CHUNK, _K), jnp.float32),
            pltpu.VMEM((_CHUNK, _K), jnp.float32),
            pltpu.VMEM((_CHUNK, _K), jnp.float32),
            pltpu.VMEM((_CHUNK, _K), jnp.float32),
            pltpu.VMEM((_BPW,), jnp.float32),
            pltpu.VMEM((384,), jnp.float32),
            pltpu.SemaphoreType.DMA,
            pltpu.SemaphoreType.DMA,
            pltpu.SemaphoreType.DMA,
            pltpu.SemaphoreType.DMA,
            pltpu.SemaphoreType.DMA,
            pltpu.SemaphoreType.DMA,
            pltpu.SemaphoreType.DMA,
        ],
    )(_sc_body)
    return sc(uid, iid, W, U, vc_flat)


# speedup vs baseline: 1.0879x; 1.0233x over previous
import functools

import jax
import jax.numpy as jnp
from jax import lax
from jax.experimental import pallas as pl
from jax.experimental.pallas import tpu as pltpu
from jax.experimental.pallas import tpu_sc as plsc

_B = 16384
_K = 128
_H = 256
_NC = 2
_NS = 16
_NW = _NC * _NS
_BPW = _B // _NW
_CHUNK = 128
_NCHUNK = _BPW // _CHUNK


def _vc_body(w1_ref, w2_ref, b1_ref, b2_ref, out_ref):
    v = jnp.dot(w2_ref[...], w1_ref[...], preferred_element_type=jnp.float32)
    c = jnp.sum(w2_ref[...] * b1_ref[...]) + b2_ref[0, 0]
    out_ref[pl.ds(0, 256)] = v[0]
    out_ref[pl.ds(256, 128)] = jnp.full((128,), c, jnp.float32)


def _sc_body(uid_hbm, iid_hbm, w_hbm, u_hbm, vc_hbm, out_hbm,
             uid_v, iid_v, wbuf0, ubuf0, wbuf1, ubuf1, wbuf2, ubuf2,
             obuf, vbuf,
             sem_w0, sem_u0, sem_w1, sem_u1, sem_w2, sem_u2, sem_p):
    wid = lax.axis_index("s") * _NC + lax.axis_index("c")
    base = wid * _BPW
    c_uid = pltpu.async_copy(uid_hbm.at[pl.ds(base, _BPW)], uid_v, sem_w2)
    c_iid = pltpu.async_copy(iid_hbm.at[pl.ds(base, _BPW)], iid_v, sem_u2)
    c_vc = pltpu.async_copy(vc_hbm, vbuf, sem_p)
    c_uid.wait()
    c_iid.wait()
    lanes = lax.iota(jnp.int32, 16)
    perms = [lanes ^ d for d in (1, 2, 4, 8)]
    masks = [(lanes & d) == 0 for d in (1, 2, 4, 8)]
    dnums = lax.GatherDimensionNumbers(
        offset_dims=(), collapsed_slice_dims=(0,), start_index_map=(0,))

    def shuf(x, idx):
        return lax.gather(x, idx[:, None], dnums, (1,),
                          mode=lax.GatherScatterMode.PROMISE_IN_BOUNDS)

    bufs = [(wbuf0, ubuf0, sem_w0, sem_u0), (wbuf1, ubuf1, sem_w1, sem_u1),
            (wbuf2, ubuf2, sem_w2, sem_u2)]

    chunks = [(0, 64), (64, 128), (192, 128), (320, 128), (448, 64)]

    def fire(k):
        off, size = chunks[k]
        wb, ub, sw, su = bufs[k % 3]
        cw = pltpu.async_copy(
            w_hbm.at[uid_v.at[pl.ds(off, size)]], wb.at[pl.ds(0, size)], sw)
        cu = pltpu.async_copy(
            u_hbm.at[iid_v.at[pl.ds(off, size)]], ub.at[pl.ds(0, size)], su)
        return cw, cu

    def compute(k):
        off, size = chunks[k]
        wb, ub, _, _ = bufs[k % 3]

        def group(t, inner):
            def jstep(j, accs):
                cw = vbuf[pl.ds(16 * j, 16)]
                cu = vbuf[pl.ds(128 + 16 * j, 16)]
                return tuple(
                    accs[i]
                    + wb[t * 16 + i, pl.ds(16 * j, 16)] * cw
                    + ub[t * 16 + i, pl.ds(16 * j, 16)] * cu
                    for i in range(16))

            zero = jnp.zeros((16,), jnp.float32)
            accs = lax.fori_loop(0, 8, jstep, (zero,) * 16)

            vals = list(accs)
            for p, m in zip(perms, masks):
                vals = [jnp.where(m, a + shuf(a, p), b + shuf(b, p))
                        for a, b in zip(vals[::2], vals[1::2])]
            outv = jnp.clip(vals[0] + cval, 0.5, 5.0)
            obuf[pl.ds(off + t * 16, 16)] = outv
            return inner

        lax.fori_loop(0, size // 16, group, 0)

    pending = [fire(0), fire(1), fire(2)]
    c_vc.wait()
    cval = vbuf[pl.ds(256, 16)][0]
    outs = []
    for k in range(len(chunks)):
        cw, cu = pending[k]
        cw.wait()
        cu.wait()
        compute(k)
        if k + 3 < len(chunks):
            pending.append(fire(k + 3))
        off, size = chunks[k]
        outs.append(pltpu.async_copy(
            obuf.at[pl.ds(off, size)],
            out_hbm.at[pl.ds(base + off, size)], sem_p))
    for c in outs:
        c.wait()


def kernel(user_ids, item_ids, W, U, W1, b1, W2, b2):
    uid = user_ids.astype(jnp.int32)
    iid = item_ids.astype(jnp.int32)

    vc_flat = pl.pallas_call(
        _vc_body,
        out_shape=jax.ShapeDtypeStruct((384,), jnp.float32),
    )(W1, W2, b1.reshape(1, _H), b2.reshape(1, 1))

    sc = functools.partial(
        pl.kernel,
        mesh=plsc.VectorSubcoreMesh(core_axis_name="c", subcore_axis_name="s"),
        out_type=jax.ShapeDtypeStruct((_B,), jnp.float32),
        scratch_types=[
            pltpu.VMEM((_BPW,), jnp.int32),
            pltpu.VMEM((_BPW,), jnp.int32),
            pltpu.VMEM((_CHUNK, _K), jnp.float32),
            pltpu.VMEM((_CHUNK, _K), jnp.float32),
            pltpu.VMEM((_CHUNK, _K), jnp.float32),
            pltpu.VMEM((_CHUNK, _K), jnp.float32),
            pltpu.VMEM((_CHUNK, _K), jnp.float32),
            pltpu.VMEM((_CHUNK, _K), jnp.float32),
            pltpu.VMEM((_BPW,), jnp.float32),
            pltpu.VMEM((384,), jnp.float32),
            pltpu.SemaphoreType.DMA,
            pltpu.SemaphoreType.DMA,
            pltpu.SemaphoreType.DMA,
            pltpu.SemaphoreType.DMA,
            pltpu.SemaphoreType.DMA,
            pltpu.SemaphoreType.DMA,
            pltpu.SemaphoreType.DMA,
        ],
    )(_sc_body)
    return sc(uid, iid, W, U, vc_flat)


# speedup vs baseline: 1.0906x; 1.0025x over previous
import functools

import jax
import jax.numpy as jnp
from jax import lax
from jax.experimental import pallas as pl
from jax.experimental.pallas import tpu as pltpu
from jax.experimental.pallas import tpu_sc as plsc

_B = 16384
_K = 128
_H = 256
_NC = 2
_NS = 16
_NW = _NC * _NS
_BPW = _B // _NW
_CHUNK = 128
_NCHUNK = _BPW // _CHUNK


def _vc_body(w1_ref, w2_ref, b1_ref, b2_ref, out_ref):
    v = jnp.dot(w2_ref[...], w1_ref[...], preferred_element_type=jnp.float32)
    c = jnp.sum(w2_ref[...] * b1_ref[...]) + b2_ref[0, 0]
    out_ref[pl.ds(0, 256)] = v[0]
    out_ref[pl.ds(256, 128)] = jnp.full((128,), c, jnp.float32)


def _sc_body(uid_hbm, iid_hbm, w_hbm, u_hbm, vc_hbm, out_hbm,
             uid_v, iid_v, wbuf0, ubuf0, wbuf1, ubuf1, wbuf2, ubuf2,
             obuf, vbuf,
             sem_w0, sem_u0, sem_w1, sem_u1, sem_w2, sem_u2, sem_p):
    wid = lax.axis_index("s") * _NC + lax.axis_index("c")
    base = wid * _BPW
    c_uid = pltpu.async_copy(uid_hbm.at[pl.ds(base, _BPW)], uid_v, sem_w2)
    c_iid = pltpu.async_copy(iid_hbm.at[pl.ds(base, _BPW)], iid_v, sem_u2)
    c_vc = pltpu.async_copy(vc_hbm, vbuf, sem_p)
    c_uid.wait()
    c_iid.wait()
    lanes = lax.iota(jnp.int32, 16)
    perms = [lanes ^ d for d in (1, 2, 4, 8)]
    masks = [(lanes & d) == 0 for d in (1, 2, 4, 8)]
    dnums = lax.GatherDimensionNumbers(
        offset_dims=(), collapsed_slice_dims=(0,), start_index_map=(0,))

    def shuf(x, idx):
        return lax.gather(x, idx[:, None], dnums, (1,),
                          mode=lax.GatherScatterMode.PROMISE_IN_BOUNDS)

    bufs = [(wbuf0, ubuf0, sem_w0, sem_u0), (wbuf1, ubuf1, sem_w1, sem_u1),
            (wbuf2, ubuf2, sem_w2, sem_u2)]

    chunks = [(0, 64), (64, 64), (128, 128), (256, 128), (384, 64), (448, 64)]

    def fire(k):
        off, size = chunks[k]
        wb, ub, sw, su = bufs[k % 3]
        cw = pltpu.async_copy(
            w_hbm.at[uid_v.at[pl.ds(off, size)]], wb.at[pl.ds(0, size)], sw)
        cu = pltpu.async_copy(
            u_hbm.at[iid_v.at[pl.ds(off, size)]], ub.at[pl.ds(0, size)], su)
        return cw, cu

    def compute(k):
        off, size = chunks[k]
        wb, ub, _, _ = bufs[k % 3]

        def group(t, inner):
            def jstep(j, accs):
                cw = vbuf[pl.ds(16 * j, 16)]
                cu = vbuf[pl.ds(128 + 16 * j, 16)]
                return tuple(
                    accs[i]
                    + wb[t * 16 + i, pl.ds(16 * j, 16)] * cw
                    + ub[t * 16 + i, pl.ds(16 * j, 16)] * cu
                    for i in range(16))

            zero = jnp.zeros((16,), jnp.float32)
            accs = lax.fori_loop(0, 8, jstep, (zero,) * 16)

            vals = list(accs)
            for p, m in zip(perms, masks):
                vals = [jnp.where(m, a + shuf(a, p), b + shuf(b, p))
                        for a, b in zip(vals[::2], vals[1::2])]
            outv = jnp.clip(vals[0] + cval, 0.5, 5.0)
            obuf[pl.ds(off + t * 16, 16)] = outv
            return inner

        lax.fori_loop(0, size // 16, group, 0)

    pending = [fire(0), fire(1), fire(2)]
    c_vc.wait()
    cval = vbuf[pl.ds(256, 16)][0]
    outs = []
    for k in range(len(chunks)):
        cw, cu = pending[k]
        cw.wait()
        cu.wait()
        compute(k)
        if k + 3 < len(chunks):
            pending.append(fire(k + 3))
        off, size = chunks[k]
        outs.append(pltpu.async_copy(
            obuf.at[pl.ds(off, size)],
            out_hbm.at[pl.ds(base + off, size)], sem_p))
    for c in outs:
        c.wait()


def kernel(user_ids, item_ids, W, U, W1, b1, W2, b2):
    uid = user_ids.astype(jnp.int32)
    iid = item_ids.astype(jnp.int32)

    vc_flat = pl.pallas_call(
        _vc_body,
        out_shape=jax.ShapeDtypeStruct((384,), jnp.float32),
    )(W1, W2, b1.reshape(1, _H), b2.reshape(1, 1))

    sc = functools.partial(
        pl.kernel,
        mesh=plsc.VectorSubcoreMesh(core_axis_name="c", subcore_axis_name="s"),
        out_type=jax.ShapeDtypeStruct((_B,), jnp.float32),
        scratch_types=[
            pltpu.VMEM((_BPW,), jnp.int32),
            pltpu.VMEM((_BPW,), jnp.int32),
            pltpu.VMEM((_CHUNK, _K), jnp.float32),
            pltpu.VMEM((_CHUNK, _K), jnp.float32),
            pltpu.VMEM((_CHUNK, _K), jnp.float32),
            pltpu.VMEM((_CHUNK, _K), jnp.float32),
            pltpu.VMEM((_CHUNK, _K), jnp.float32),
            pltpu.VMEM((_CHUNK, _K), jnp.float32),
            pltpu.VMEM((_BPW,), jnp.float32),
            pltpu.VMEM((384,), jnp.float32),
            pltpu.SemaphoreType.DMA,
            pltpu.SemaphoreType.DMA,
            pltpu.SemaphoreType.DMA,
            pltpu.SemaphoreType.DMA,
            pltpu.SemaphoreType.DMA,
            pltpu.SemaphoreType.DMA,
            pltpu.SemaphoreType.DMA,
        ],
    )(_sc_body)
    return sc(uid, iid, W, U, vc_flat)
